# Initial kernel scaffold; baseline (speedup 1.0000x reference)
import jax
import jax.numpy as jnp
from jax.experimental import pallas as pl


def kernel(x, edge_index, edge_attr, W1, b1, W2, b2):
    raise NotImplementedError("write your pallas kernel here")


# speedup vs baseline: 13.7103x; 13.7103x over previous
import functools

import jax
import jax.numpy as jnp
from jax import lax
from jax.experimental import pallas as pl
from jax.experimental.pallas import tpu as pltpu
from jax.experimental.pallas import tpu_sc as plsc

N = 100000
E = 1600000
HID = 32
HALF = 16
NSC = 2
NT = 16
CHUNK = 2000
SUB = 125
NSUB = CHUNK // SUB
GROUPS = CHUNK // 16
EPT_MSG = E // NT
NCH_MSG = EPT_MSG // CHUNK
EPSC_DEG = E // NSC
EPT_DEG = EPSC_DEG // NT
NCH_DEG = EPT_DEG // CHUNK
HN = 50000
TRASH = 1024
ACC_ROWS = 51200
TBASE = 50176
APT = ACC_ROWS // NT
BN = 4000
GRID = N // BN

_mesh = plsc.VectorSubcoreMesh(core_axis_name="c", subcore_axis_name="s")


def _zero_accum(zeros_hbm, accum, t):
    pltpu.sync_copy(zeros_hbm.at[pl.ds(t * APT, APT)],
                    accum.at[pl.ds(t * APT, APT)])


def _drain_accum(accum, out_hbm, c, t):
    pltpu.sync_copy(accum.at[pl.ds(t * APT, APT)],
                    out_hbm.at[c, pl.ds(t * APT, APT)])


@functools.partial(
    pl.kernel,
    out_type=jax.ShapeDtypeStruct((NSC, ACC_ROWS, HALF), jnp.float32),
    mesh=_mesh,
    compiler_params=pltpu.CompilerParams(use_tc_tiling_on_sc=False),
    scratch_types=[
        pltpu.VMEM((NSUB, SUB), jnp.int32),
        pltpu.VMEM((CHUNK,), jnp.float32),
        pltpu.VMEM((CHUNK, HALF), jnp.float32),
        pltpu.VMEM_SHARED((ACC_ROWS, HALF), jnp.float32),
    ],
)
def _deg_pass(col2d, ew, zeros, out, col_v, ew_v, dbuf, accum):
    c = lax.axis_index("c")
    t = lax.axis_index("s")
    _zero_accum(zeros, accum, t)
    plsc.subcore_barrier()
    iot = lax.iota(jnp.int32, 16)

    def chunk_body(i, carry):
        ebase = c * EPSC_DEG + t * EPT_DEG + i * CHUNK
        rbase = c * (EPSC_DEG // SUB) + t * (EPT_DEG // SUB) + i * NSUB
        pltpu.sync_copy(ew.at[pl.ds(ebase, CHUNK)], ew_v)
        pltpu.sync_copy(col2d.at[pl.ds(rbase, NSUB)], col_v)

        def g_body(g, carry2):
            ewv = ew_v[pl.ds(g * 16, 16)]
            for cc in range(16):
                dbuf[g * 16 + cc, :] = jnp.where(iot == cc, ewv, 0.0)
            return carry2

        lax.fori_loop(0, GROUPS, g_body, 0)
        for s in range(NSUB):
            pltpu.sync_copy(dbuf.at[pl.ds(s * SUB, SUB)],
                            accum.at[col_v.at[s]], add=True)
        return carry

    lax.fori_loop(0, NCH_DEG, chunk_body, 0)
    plsc.subcore_barrier()
    _drain_accum(accum, out, c, t)


@functools.partial(
    pl.kernel,
    out_type=jax.ShapeDtypeStruct((NSC, ACC_ROWS, HALF), jnp.float32),
    mesh=_mesh,
    compiler_params=pltpu.CompilerParams(use_tc_tiling_on_sc=False),
    scratch_types=[
        pltpu.VMEM((NSUB, SUB), jnp.int32),
        pltpu.VMEM((NSUB, SUB), jnp.int32),
        pltpu.VMEM((CHUNK,), jnp.float32),
        pltpu.VMEM((CHUNK, HALF), jnp.float32),
        pltpu.VMEM_SHARED((ACC_ROWS, HALF), jnp.float32),
        pltpu.SemaphoreType.DMA,
    ],
)
def _msg_pass(xws0, xws1, row2d, col2d, ew, zeros, out,
              row_v, col_v, ew_v, gbuf, accum, sem):
    c = lax.axis_index("c")
    t = lax.axis_index("s")
    _zero_accum(zeros, accum, t)
    plsc.subcore_barrier()
    iot = lax.iota(jnp.int32, 16)

    def chunk_body(i, carry):
        ebase = t * EPT_MSG + i * CHUNK
        rbase = t * (EPT_MSG // SUB) + i * NSUB
        pltpu.sync_copy(ew.at[pl.ds(ebase, CHUNK)], ew_v)
        pltpu.sync_copy(row2d.at[pl.ds(rbase, NSUB)], row_v)
        pltpu.sync_copy(col2d.at[pl.ds(rbase, NSUB)], col_v)

        @pl.when(c == 0)
        def _():
            cps = [pltpu.async_copy(xws0.at[row_v.at[s]],
                                    gbuf.at[pl.ds(s * SUB, SUB)], sem)
                   for s in range(NSUB)]
            for cp in cps:
                cp.wait()

        @pl.when(c == 1)
        def _():
            cps = [pltpu.async_copy(xws1.at[row_v.at[s]],
                                    gbuf.at[pl.ds(s * SUB, SUB)], sem)
                   for s in range(NSUB)]
            for cp in cps:
                cp.wait()

        def g_body(g, carry2):
            ewv = ew_v[pl.ds(g * 16, 16)]
            base = g * 16
            for cc in range(16):
                splat = lax.gather(
                    ewv, jnp.full((16, 1), cc, jnp.int32),
                    lax.GatherDimensionNumbers(offset_dims=(),
                                               collapsed_slice_dims=(0,),
                                               start_index_map=(0,)),
                    (1,), mode=lax.GatherScatterMode.PROMISE_IN_BOUNDS)
                gbuf[base + cc, :] = gbuf[base + cc, :] * splat
            return carry2

        lax.fori_loop(0, GROUPS, g_body, 0)
        for s in range(NSUB):
            pltpu.sync_copy(gbuf.at[pl.ds(s * SUB, SUB)],
                            accum.at[col_v.at[s]], add=True)
        return carry

    lax.fori_loop(0, NCH_MSG, chunk_body, 0)
    plsc.subcore_barrier()
    _drain_accum(accum, out, c, t)


def _dense1_body(x_ref, w1_ref, degp_ref, xw1_ref, dis_ref, xwsp_ref):
    xw = jnp.dot(x_ref[...], w1_ref[...], preferred_element_type=jnp.float32)
    deg = 1.0 + jnp.sum(degp_ref[...], axis=(0, 2))
    dis = lax.rsqrt(deg)
    xw1_ref[...] = xw
    dis_ref[...] = dis[:, None]
    xws = xw * dis[:, None]
    xwsp_ref[...] = jnp.stack([xws[:, :HALF], xws[:, HALF:]])


def _dense2_body(a_ref, xw1_ref, dis_ref, w2_ref, b1_ref, xw2_ref, xwsp_ref):
    dis = dis_ref[:, 0]
    acat = jnp.concatenate([a_ref[0], a_ref[1]], axis=1)
    h = jax.nn.relu(dis[:, None] * acat + (dis * dis)[:, None] * xw1_ref[...]
                    + b1_ref[...])
    xw2 = jnp.dot(h, w2_ref[...], preferred_element_type=jnp.float32)
    xw2_ref[...] = xw2
    xws = xw2 * dis[:, None]
    xwsp_ref[...] = jnp.stack([xws[:, :HALF], xws[:, HALF:]])


def _dense3_body(a_ref, xw2_ref, dis_ref, b2_ref, out_ref):
    dis = dis_ref[:, 0]
    acat = jnp.concatenate([a_ref[0], a_ref[1]], axis=1)
    out_ref[...] = jax.nn.relu(dis[:, None] * acat
                               + (dis * dis)[:, None] * xw2_ref[...]
                               + b2_ref[...])


_spec_n32 = pl.BlockSpec((BN, HID), lambda i: (i, 0))
_spec_planes = pl.BlockSpec((NSC, BN, HALF), lambda i: (0, i, 0))
_spec_dis = pl.BlockSpec((BN, 1), lambda i: (i, 0))

_dense1 = pl.pallas_call(
    _dense1_body,
    grid=(GRID,),
    in_specs=[
        pl.BlockSpec((BN, 4), lambda i: (i, 0)),
        pl.BlockSpec((4, HID), lambda i: (0, 0)),
        _spec_planes,
    ],
    out_specs=[_spec_n32, _spec_dis, _spec_planes],
    out_shape=[
        jax.ShapeDtypeStruct((N, HID), jnp.float32),
        jax.ShapeDtypeStruct((N, 1), jnp.float32),
        jax.ShapeDtypeStruct((NSC, N, HALF), jnp.float32),
    ],
)

_dense2 = pl.pallas_call(
    _dense2_body,
    grid=(GRID,),
    in_specs=[
        _spec_planes,
        _spec_n32,
        _spec_dis,
        pl.BlockSpec((HID, HID), lambda i: (0, 0)),
        pl.BlockSpec((HID,), lambda i: (0,)),
    ],
    out_specs=[_spec_n32, _spec_planes],
    out_shape=[
        jax.ShapeDtypeStruct((N, HID), jnp.float32),
        jax.ShapeDtypeStruct((NSC, N, HALF), jnp.float32),
    ],
)

_dense3 = pl.pallas_call(
    _dense3_body,
    grid=(GRID,),
    in_specs=[
        _spec_planes,
        _spec_n32,
        _spec_dis,
        pl.BlockSpec((HID,), lambda i: (0,)),
    ],
    out_specs=_spec_n32,
    out_shape=jax.ShapeDtypeStruct((N, HID), jnp.float32),
)


def kernel(x, edge_index, edge_attr, W1, b1, W2, b2):
    row = edge_index[0].astype(jnp.int32)
    col = edge_index[1].astype(jnp.int32)
    row2d = row.reshape(E // SUB, SUB)
    trash = TBASE + (jnp.arange(E, dtype=jnp.int32) % TRASH)
    colA2d = jnp.where(col < HN, col, trash).reshape(E // SUB, SUB)
    colB2d = jnp.where(col >= HN, col - HN, trash).reshape(E // SUB, SUB)
    ew = edge_attr.astype(jnp.float32)
    zeros = jnp.zeros((ACC_ROWS, HALF), jnp.float32)

    dA = _deg_pass(colA2d, ew, zeros)
    dB = _deg_pass(colB2d, ew, zeros)
    degp = jnp.concatenate([dA[:, :HN], dB[:, :HN]], axis=1)
    xw1, dis, xwsp = _dense1(x.astype(jnp.float32), W1, degp)
    a1A = _msg_pass(xwsp[0], xwsp[1], row2d, colA2d, ew, zeros)
    a1B = _msg_pass(xwsp[0], xwsp[1], row2d, colB2d, ew, zeros)
    a1 = jnp.concatenate([a1A[:, :HN], a1B[:, :HN]], axis=1)
    xw2, xws2p = _dense2(a1, xw1, dis, W2, b1)
    a2A = _msg_pass(xws2p[0], xws2p[1], row2d, colA2d, ew, zeros)
    a2B = _msg_pass(xws2p[0], xws2p[1], row2d, colB2d, ew, zeros)
    a2 = jnp.concatenate([a2A[:, :HN], a2B[:, :HN]], axis=1)
    return _dense3(a2, xw2, dis, b2)
